# baseline (speedup 1.0000x reference)
import functools

import jax
import jax.numpy as jnp
from jax import lax
from jax.experimental import pallas as pl
from jax.experimental.pallas import tpu as pltpu


def _fused_kernel(tf_ref, ri_row_ref, ri_col_ref, msa_ref,
                  pk_w_ref, pk_b_ref, w_rel_ref, w_msa_ref,
                  msa_out_ref, pair_out_ref, *, relpos_k, tf_dim, msa_dim,
                  c_z, c_m, num_bins):
    n = tf_ref.shape[1]
    ts = msa_ref.shape[2]
    ti = pair_out_ref.shape[1]

    tf = tf_ref[0]
    w_zsum = pk_w_ref[:, :c_z]
    w_tfm = pk_w_ref[:, c_z:c_z + c_m]
    b_zsum = pk_b_ref[:, :c_z]
    b_tm = pk_b_ref[:, c_z:c_z + c_m]

    tf_m = jnp.dot(tf, w_tfm, preferred_element_type=jnp.float32) + b_tm
    msa = msa_ref[0].astype(jnp.bfloat16)
    l1 = lax.dot_general(msa, w_msa_ref[...].astype(jnp.bfloat16),
                         (((0,), (0,)), ((), ())),
                         preferred_element_type=jnp.float32)
    msa_out_ref[0] = (l1 + tf_m[None, :, :]).astype(msa_out_ref.dtype)

    bias = jnp.dot(tf, w_zsum, preferred_element_type=jnp.float32) + b_zsum
    ri_i = ri_row_ref[0]
    ri_j = ri_col_ref[0]
    idx = jnp.clip(ri_i - ri_j + relpos_k, 0, num_bins - 1)
    lane = lax.broadcasted_iota(jnp.int32, (ti, n, num_bins), 2)
    one_hot = (lane == idx[:, :, None]).astype(jnp.bfloat16)
    relpos = jnp.dot(one_hot.reshape(ti * n, num_bins),
                     w_rel_ref[...].astype(jnp.bfloat16),
                     preferred_element_type=jnp.float32)
    pair_out_ref[0] = (relpos.reshape(ti, n, c_z)
                       + bias[None, :, :]).astype(pair_out_ref.dtype)


def _pick_steps(S, N):
    for g in (16, 8, 4, 2):
        if S % g == 0 and N % g == 0 and (N // g) % 8 == 0:
            return g
    return 1


def kernel(target_feat, residue_index, msa_feat, w_zsum, b_zsum, w_tfm, b_tfm,
           w_rel, w_msa, b_msa):
    B, N, tf_dim = target_feat.shape
    S, msa_dim = msa_feat.shape[1], msa_feat.shape[3]
    num_bins, c_z = w_rel.shape
    c_m = w_msa.shape[1]
    relpos_k = (num_bins - 1) // 2

    g = _pick_steps(S, N)
    ts, ti = S // g, N // g

    ri = residue_index.astype(jnp.int32)
    ri_row = ri.reshape(B, N, 1)
    ri_col = ri.reshape(B, 1, N)

    msa_t = msa_feat.transpose(0, 3, 1, 2)

    pk_w = jnp.concatenate([w_zsum, w_tfm], axis=1)
    pk_b = jnp.concatenate([b_zsum, b_tfm + b_msa], axis=1)

    body = functools.partial(_fused_kernel, relpos_k=relpos_k, tf_dim=tf_dim,
                             msa_dim=msa_dim, c_z=c_z, c_m=c_m,
                             num_bins=num_bins)
    msa_out, pair_out = pl.pallas_call(
        body,
        out_shape=(jax.ShapeDtypeStruct((B, S, N, c_m), jnp.float32),
                   jax.ShapeDtypeStruct((B, N, N, c_z), jnp.float32)),
        grid=(B, g),
        in_specs=[
            pl.BlockSpec((1, N, tf_dim), lambda b, s: (b, 0, 0)),
            pl.BlockSpec((1, ti, 1), lambda b, s: (b, s, 0)),
            pl.BlockSpec((1, 1, N), lambda b, s: (b, 0, 0)),
            pl.BlockSpec((1, msa_dim, ts, N), lambda b, s: (b, 0, s, 0)),
            pl.BlockSpec((tf_dim, c_z + c_m), lambda b, s: (0, 0)),
            pl.BlockSpec((1, c_z + c_m), lambda b, s: (0, 0)),
            pl.BlockSpec((num_bins, c_z), lambda b, s: (0, 0)),
            pl.BlockSpec((msa_dim, c_m), lambda b, s: (0, 0)),
        ],
        out_specs=(pl.BlockSpec((1, ts, N, c_m), lambda b, s: (b, s, 0, 0)),
                   pl.BlockSpec((1, ti, N, c_z), lambda b, s: (b, s, 0, 0))),
        compiler_params=pltpu.CompilerParams(
            dimension_semantics=("parallel", "parallel"),
            vmem_limit_bytes=48 * 1024 * 1024),
    )(target_feat, ri_row, ri_col, msa_t, pk_w, pk_b, w_rel, w_msa)
    return msa_out, pair_out


# speedup vs baseline: 1.0129x; 1.0129x over previous
import functools

import jax
import jax.numpy as jnp
from jax import lax
from jax.experimental import pallas as pl
from jax.experimental.pallas import tpu as pltpu


def _fused_kernel(tf_ref, ri_row_ref, ri_col_ref, msa_ref,
                  pk_w_ref, pk_b_ref, pk_r_ref,
                  msa_out_ref, pair_out_ref, *, relpos_k, tf_dim, msa_dim,
                  c_z, c_m, num_bins):
    n = tf_ref.shape[1]
    ts = msa_ref.shape[2]
    ti = pair_out_ref.shape[1]

    tf = tf_ref[0]
    w_zsum = pk_w_ref[:, :c_z]
    w_tfm = pk_w_ref[:, c_z:c_z + c_m]
    b_zsum = pk_b_ref[:, :c_z]
    b_tm = pk_b_ref[:, c_z:c_z + c_m]
    pk_r = pk_r_ref[...].astype(jnp.bfloat16)
    w_rel = pk_r[:, :c_z]
    w_msa = pk_r[:msa_dim, c_z:c_z + c_m]

    tf_m = jnp.dot(tf, w_tfm, preferred_element_type=jnp.float32) + b_tm
    msa = msa_ref[0].astype(jnp.bfloat16)
    l1 = lax.dot_general(msa, w_msa, (((0,), (0,)), ((), ())),
                         preferred_element_type=jnp.float32)
    msa_out_ref[0] = (l1 + tf_m[None, :, :]).astype(msa_out_ref.dtype)

    bias = jnp.dot(tf, w_zsum, preferred_element_type=jnp.float32) + b_zsum
    ri_i = ri_row_ref[0]
    ri_j = ri_col_ref[0]
    idx = jnp.clip(ri_i - ri_j + relpos_k, 0, num_bins - 1)
    lane = lax.broadcasted_iota(jnp.int32, (ti, n, num_bins), 2)
    one_hot = (lane == idx[:, :, None]).astype(jnp.bfloat16)
    relpos = jnp.dot(one_hot.reshape(ti * n, num_bins), w_rel,
                     preferred_element_type=jnp.float32)
    pair_out_ref[0] = (relpos.reshape(ti, n, c_z)
                       + bias[None, :, :]).astype(pair_out_ref.dtype)


def _pick_steps(S, N):
    for g in (16, 8, 4, 2):
        if S % g == 0 and N % g == 0 and (N // g) % 8 == 0:
            return g
    return 1


def kernel(target_feat, residue_index, msa_feat, w_zsum, b_zsum, w_tfm, b_tfm,
           w_rel, w_msa, b_msa):
    B, N, tf_dim = target_feat.shape
    S, msa_dim = msa_feat.shape[1], msa_feat.shape[3]
    num_bins, c_z = w_rel.shape
    c_m = w_msa.shape[1]
    relpos_k = (num_bins - 1) // 2

    g = _pick_steps(S, N)
    ts, ti = S // g, N // g

    ri = residue_index.astype(jnp.int32)
    ri_row = ri.reshape(B, N, 1)
    ri_col = ri.reshape(B, 1, N)

    msa_t = msa_feat.transpose(0, 3, 1, 2)

    pk_w = jnp.concatenate([w_zsum, w_tfm], axis=1)
    pk_b = jnp.concatenate([b_zsum, b_tfm + b_msa], axis=1)
    pk_r = jnp.concatenate(
        [w_rel, jnp.pad(w_msa, ((0, num_bins - msa_dim), (0, 0)))],
        axis=1)

    body = functools.partial(_fused_kernel, relpos_k=relpos_k, tf_dim=tf_dim,
                             msa_dim=msa_dim, c_z=c_z, c_m=c_m,
                             num_bins=num_bins)
    msa_out, pair_out = pl.pallas_call(
        body,
        out_shape=(jax.ShapeDtypeStruct((B, S, N, c_m), jnp.float32),
                   jax.ShapeDtypeStruct((B, N, N, c_z), jnp.float32)),
        grid=(B, g),
        in_specs=[
            pl.BlockSpec((1, N, tf_dim), lambda b, s: (b, 0, 0)),
            pl.BlockSpec((1, ti, 1), lambda b, s: (b, s, 0)),
            pl.BlockSpec((1, 1, N), lambda b, s: (b, 0, 0)),
            pl.BlockSpec((1, msa_dim, ts, N), lambda b, s: (b, 0, s, 0)),
            pl.BlockSpec((tf_dim, c_z + c_m), lambda b, s: (0, 0)),
            pl.BlockSpec((1, c_z + c_m), lambda b, s: (0, 0)),
            pl.BlockSpec((num_bins, c_z + c_m), lambda b, s: (0, 0)),
        ],
        out_specs=(pl.BlockSpec((1, ts, N, c_m), lambda b, s: (b, s, 0, 0)),
                   pl.BlockSpec((1, ti, N, c_z), lambda b, s: (b, s, 0, 0))),
        compiler_params=pltpu.CompilerParams(
            dimension_semantics=("parallel", "parallel"),
            vmem_limit_bytes=48 * 1024 * 1024),
    )(target_feat, ri_row, ri_col, msa_t, pk_w, pk_b, pk_r)
    return msa_out, pair_out


# speedup vs baseline: 1.0134x; 1.0004x over previous
import functools

import jax
import jax.numpy as jnp
from jax import lax
from jax.experimental import pallas as pl
from jax.experimental.pallas import tpu as pltpu


def _fused_kernel(ri_row_ref, ri_col_ref, msa_ref, sp_ref,
                  msa_out_ref, pair_out_ref, *, relpos_k, tf_dim, msa_dim,
                  c_z, c_m, num_bins):
    n = ri_col_ref.shape[2]
    ts = msa_ref.shape[2]
    ti = pair_out_ref.shape[1]

    tf_t = sp_ref[0:tf_dim, 0:n]
    w_zsum = sp_ref[24:24 + tf_dim, :c_z]
    w_tfm = sp_ref[24:24 + tf_dim, c_z:c_z + c_m]
    b_zsum = sp_ref[48:49, :c_z]
    b_tm = sp_ref[48:49, c_z:c_z + c_m]
    w_rel = sp_ref[56:56 + num_bins, :c_z].astype(jnp.bfloat16)
    w_msa = sp_ref[56:56 + msa_dim, c_z:c_z + c_m].astype(jnp.bfloat16)

    tf_m = lax.dot_general(tf_t, w_tfm, (((0,), (0,)), ((), ())),
                           preferred_element_type=jnp.float32) + b_tm
    msa = msa_ref[0].astype(jnp.bfloat16)
    l1 = lax.dot_general(msa, w_msa, (((0,), (0,)), ((), ())),
                         preferred_element_type=jnp.float32)
    msa_out_ref[0] = (l1 + tf_m[None, :, :]).astype(msa_out_ref.dtype)

    bias = lax.dot_general(tf_t, w_zsum, (((0,), (0,)), ((), ())),
                           preferred_element_type=jnp.float32) + b_zsum
    ri_i = ri_row_ref[0]
    ri_j = ri_col_ref[0]
    idx = jnp.clip(ri_i - ri_j + relpos_k, 0, num_bins - 1)
    lane = lax.broadcasted_iota(jnp.int32, (ti, n, num_bins), 2)
    one_hot = (lane == idx[:, :, None]).astype(jnp.bfloat16)
    relpos = jnp.dot(one_hot.reshape(ti * n, num_bins), w_rel,
                     preferred_element_type=jnp.float32)
    pair_out_ref[0] = (relpos.reshape(ti, n, c_z)
                       + bias[None, :, :]).astype(pair_out_ref.dtype)


def _pick_steps(S, N):
    for g in (16, 8, 4, 2):
        if S % g == 0 and N % g == 0 and (N // g) % 8 == 0:
            return g
    return 1


def kernel(target_feat, residue_index, msa_feat, w_zsum, b_zsum, w_tfm, b_tfm,
           w_rel, w_msa, b_msa):
    B, N, tf_dim = target_feat.shape
    S, msa_dim = msa_feat.shape[1], msa_feat.shape[3]
    num_bins, c_z = w_rel.shape
    c_m = w_msa.shape[1]
    relpos_k = (num_bins - 1) // 2

    g = _pick_steps(S, N)
    ts, ti = S // g, N // g

    ri = residue_index.astype(jnp.int32)
    ri_row = ri.reshape(B, N, 1)
    ri_col = ri.reshape(B, 1, N)

    msa_t = msa_feat.transpose(0, 3, 1, 2)

    w = c_z + c_m
    tf_t = target_feat.reshape(N, tf_dim).T
    blk0 = jnp.pad(tf_t, ((0, 24 - tf_dim), (0, w - N)))
    blk1 = jnp.pad(jnp.concatenate([w_zsum, w_tfm], axis=1),
                   ((0, 24 - tf_dim), (0, 0)))
    blk2 = jnp.pad(jnp.concatenate([b_zsum, b_tfm + b_msa], axis=1),
                   ((0, 7), (0, 0)))
    blk3 = jnp.concatenate(
        [w_rel, jnp.pad(w_msa, ((0, num_bins - msa_dim), (0, 0)))],
        axis=1)
    sp = jnp.concatenate([blk0, blk1, blk2, blk3], axis=0)

    body = functools.partial(_fused_kernel, relpos_k=relpos_k, tf_dim=tf_dim,
                             msa_dim=msa_dim, c_z=c_z, c_m=c_m,
                             num_bins=num_bins)
    msa_out, pair_out = pl.pallas_call(
        body,
        out_shape=(jax.ShapeDtypeStruct((B, S, N, c_m), jnp.float32),
                   jax.ShapeDtypeStruct((B, N, N, c_z), jnp.float32)),
        grid=(B, g),
        in_specs=[
            pl.BlockSpec((1, ti, 1), lambda b, s: (b, s, 0)),
            pl.BlockSpec((1, 1, N), lambda b, s: (b, 0, 0)),
            pl.BlockSpec((1, msa_dim, ts, N), lambda b, s: (b, 0, s, 0)),
            pl.BlockSpec((56 + num_bins, c_z + c_m), lambda b, s: (0, 0)),
        ],
        out_specs=(pl.BlockSpec((1, ts, N, c_m), lambda b, s: (b, s, 0, 0)),
                   pl.BlockSpec((1, ti, N, c_z), lambda b, s: (b, s, 0, 0))),
        compiler_params=pltpu.CompilerParams(
            dimension_semantics=("parallel", "parallel"),
            vmem_limit_bytes=48 * 1024 * 1024),
    )(ri_row, ri_col, msa_t, sp)
    return msa_out, pair_out
